# baseline (speedup 1.0000x reference)
import functools

import jax
import jax.numpy as jnp
from jax import lax
from jax.experimental import pallas as pl
from jax.experimental.pallas import tpu as pltpu
from jax.experimental.pallas import tpu_sc as plsc

_NC = 2
_NS = 16
_NW = _NC * _NS
_LANES = 16


def _make_sc_kernel(B, PD, RW, NSEG, S):
    mesh = plsc.VectorSubcoreMesh(core_axis_name="c", subcore_axis_name="s")

    @functools.partial(
        pl.kernel,
        mesh=mesh,
        out_type=jax.ShapeDtypeStruct((B, PD), jnp.float32),
        scratch_types=[
            pltpu.VMEM((S,), jnp.float32),
            pltpu.VMEM((S,), jnp.float32),
        ],
    )
    def k(x_hbm, e_hbm, o_hbm, e_v, buf):
        wid = lax.axis_index("s") * _NC + lax.axis_index("c")
        r0 = wid * RW
        for c in range(NSEG):
            pltpu.sync_copy(e_hbm.at[pl.ds(c * S, S)], e_v)
            for r in range(RW):
                row = r0 + r
                pltpu.sync_copy(x_hbm.at[row, pl.ds(c * S, S)], buf)

                def body(i, _):
                    sl = pl.ds(i * _LANES, _LANES)
                    plsc.addupdate(buf.at[sl], e_v[sl])
                    return _

                lax.fori_loop(0, S // _LANES, body, None)
                pltpu.sync_copy(buf, o_hbm.at[row, pl.ds(c * S, S)])

    return k


def kernel(encoded_patches, position_embedding):
    B, P, D = encoded_patches.shape
    PD = P * D
    x2 = encoded_patches.reshape(B, PD)
    e1 = position_embedding.reshape(PD)
    RW = B // _NW
    NSEG = 8
    S = PD // NSEG
    out2 = _make_sc_kernel(B, PD, RW, NSEG, S)(x2, e1)
    return out2.reshape(B, P, D)


# speedup vs baseline: 1.1397x; 1.1397x over previous
import functools

import jax
import jax.numpy as jnp
from jax import lax
from jax.experimental import pallas as pl
from jax.experimental.pallas import tpu as pltpu
from jax.experimental.pallas import tpu_sc as plsc

_NC = 2
_NS = 16
_NW = _NC * _NS
_L = 16
_U = 8


def _make_sc_kernel(B, PD, RW, NSEG, S):
    mesh = plsc.VectorSubcoreMesh(core_axis_name="c", subcore_axis_name="s")

    @functools.partial(
        pl.kernel,
        mesh=mesh,
        out_type=jax.ShapeDtypeStruct((B, PD), jnp.float32),
        scratch_types=[
            pltpu.VMEM((S,), jnp.float32),
            pltpu.VMEM((2, S), jnp.float32),
            pltpu.SemaphoreType.DMA,
            pltpu.SemaphoreType.DMA,
        ],
    )
    def k(x_hbm, e_hbm, o_hbm, e_v, bufs, in_sem, out_sem):
        wid = lax.axis_index("s") * _NC + lax.axis_index("c")
        r0 = wid * RW

        def add_seg(buf):
            def body(i, carry):
                base = i * (_L * _U)
                for u in range(_U):
                    sl = pl.ds(base + u * _L, _L)
                    plsc.addupdate(buf.at[sl], e_v[sl])
                return carry

            lax.fori_loop(0, S // (_L * _U), body, 0)

        for c in range(NSEG):
            col = pl.ds(c * S, S)
            pltpu.sync_copy(e_hbm.at[col], e_v)
            in_d = [None] * RW
            out_d = [None] * RW
            in_d[0] = pltpu.async_copy(x_hbm.at[r0, col], bufs.at[0], in_sem)
            for r in range(RW):
                s = r % 2
                if r + 1 < RW:
                    if r >= 1:
                        out_d[r - 1].wait()
                    in_d[r + 1] = pltpu.async_copy(
                        x_hbm.at[r0 + r + 1, col], bufs.at[1 - s], in_sem
                    )
                in_d[r].wait()
                add_seg(bufs.at[s])
                out_d[r] = pltpu.async_copy(bufs.at[s], o_hbm.at[r0 + r, col], out_sem)
            out_d[RW - 2].wait()
            out_d[RW - 1].wait()

    return k


def kernel(encoded_patches, position_embedding):
    B, P, D = encoded_patches.shape
    PD = P * D
    x2 = encoded_patches.reshape(B, PD)
    e1 = position_embedding.reshape(PD)
    RW = B // _NW
    NSEG = 8
    S = PD // NSEG
    out2 = _make_sc_kernel(B, PD, RW, NSEG, S)(x2, e1)
    return out2.reshape(B, P, D)


# speedup vs baseline: 1.7679x; 1.5512x over previous
import functools

import jax
import jax.numpy as jnp
from jax import lax
from jax.experimental import pallas as pl
from jax.experimental.pallas import tpu as pltpu
from jax.experimental.pallas import tpu_sc as plsc

_NC = 2
_NS = 16
_NW = _NC * _NS
_L = 16


def _make_sc_kernel(B, PD, RW, NSEG, S):
    mesh = plsc.VectorSubcoreMesh(core_axis_name="c", subcore_axis_name="s")

    @functools.partial(
        pl.kernel,
        mesh=mesh,
        out_type=jax.ShapeDtypeStruct((B, PD), jnp.float32),
        scratch_types=[
            pltpu.VMEM((2, S), jnp.float32),
            pltpu.VMEM((2, RW, S), jnp.float32),
            pltpu.SemaphoreType.DMA,
            pltpu.SemaphoreType.DMA,
            pltpu.SemaphoreType.DMA,
        ],
    )
    def k(x_hbm, e_hbm, o_hbm, e_bufs, bufs, e_sem, in_sem, out_sem):
        wid = lax.axis_index("s") * _NC + lax.axis_index("c")
        rows = pl.ds(wid * RW, RW)

        def e_copy(c, s):
            return pltpu.async_copy(e_hbm.at[pl.ds(c * S, S)], e_bufs.at[s], e_sem)

        def in_copy(c, s):
            return pltpu.async_copy(
                x_hbm.at[rows, pl.ds(c * S, S)], bufs.at[s], in_sem
            )

        def out_copy(c, s):
            return pltpu.async_copy(
                bufs.at[s], o_hbm.at[rows, pl.ds(c * S, S)], out_sem
            )

        def add_block(buf, e_v):
            def body(i, carry):
                sl = pl.ds(i * _L, _L)
                ev = e_v[sl]
                for r in range(RW):
                    plsc.addupdate(buf.at[r, sl], ev)
                return carry

            lax.fori_loop(0, S // _L, body, 0)

        e_d = [None] * NSEG
        in_d = [None] * NSEG
        out_d = [None] * NSEG
        e_d[0] = e_copy(0, 0)
        in_d[0] = in_copy(0, 0)
        for c in range(NSEG):
            s = c % 2
            if c + 1 < NSEG:
                if c >= 1:
                    out_d[c - 1].wait()
                e_d[c + 1] = e_copy(c + 1, 1 - s)
                in_d[c + 1] = in_copy(c + 1, 1 - s)
            e_d[c].wait()
            in_d[c].wait()
            add_block(bufs.at[s], e_bufs.at[s])
            out_d[c] = out_copy(c, s)
        out_d[NSEG - 2].wait()
        out_d[NSEG - 1].wait()

    return k


def kernel(encoded_patches, position_embedding):
    B, P, D = encoded_patches.shape
    PD = P * D
    x2 = encoded_patches.reshape(B, PD)
    e1 = position_embedding.reshape(PD)
    RW = B // _NW
    NSEG = 32
    S = PD // NSEG
    out2 = _make_sc_kernel(B, PD, RW, NSEG, S)(x2, e1)
    return out2.reshape(B, P, D)


# speedup vs baseline: 2.1601x; 1.2218x over previous
import jax
import jax.numpy as jnp
from jax.experimental import pallas as pl
from jax.experimental.pallas import tpu as pltpu


def _make_stream_kernel(B, PD, CB, K, M):
    NCHUNK = B // CB

    def _stream_kernel(x_hbm, e_vmem, o_hbm, buf_in, buf_out, in_sem, out_sem):
        def in_copy(c):
            return pltpu.make_async_copy(
                x_hbm.at[pl.ds(c * CB, CB), :], buf_in.at[c % K], in_sem.at[c % K]
            )

        def out_copy(c):
            return pltpu.make_async_copy(
                buf_out.at[c % M], o_hbm.at[pl.ds(c * CB, CB), :], out_sem.at[c % M]
            )

        def start_in(c):
            pltpu.async_copy(
                x_hbm.at[pl.ds(c * CB, CB), :],
                buf_in.at[c % K],
                in_sem.at[c % K],
                priority=c % 2,
            )

        def start_out(c):
            pltpu.async_copy(
                buf_out.at[c % M],
                o_hbm.at[pl.ds(c * CB, CB), :],
                out_sem.at[c % M],
                priority=c % 2,
            )

        for c in range(min(K, NCHUNK)):
            start_in(c)
        for c in range(NCHUNK):
            in_copy(c).wait()
            if c >= M:
                out_copy(c - M).wait()
            buf_out[c % M] = buf_in[c % K] + e_vmem[...]
            start_out(c)
            if c + K < NCHUNK:
                start_in(c + K)
        for c in range(max(NCHUNK - M, 0), NCHUNK):
            out_copy(c).wait()

    return _stream_kernel


def kernel(encoded_patches, position_embedding):
    B, P, D = encoded_patches.shape
    PD = P * D
    x2 = encoded_patches.reshape(B, PD)
    e2 = position_embedding.reshape(1, PD)
    CB = 8
    K = 4
    M = 4
    out2 = pl.pallas_call(
        _make_stream_kernel(B, PD, CB, K, M),
        in_specs=[
            pl.BlockSpec(memory_space=pltpu.MemorySpace.HBM),
            pl.BlockSpec(memory_space=pltpu.MemorySpace.VMEM),
        ],
        out_specs=pl.BlockSpec(memory_space=pltpu.MemorySpace.HBM),
        out_shape=jax.ShapeDtypeStruct((B, PD), jnp.float32),
        scratch_shapes=[
            pltpu.MemorySpace.VMEM((K, CB, PD), jnp.float32),
            pltpu.MemorySpace.VMEM((M, CB, PD), jnp.float32),
            pltpu.SemaphoreType.DMA((K,)),
            pltpu.SemaphoreType.DMA((M,)),
        ],
    )(x2, e2)
    return out2.reshape(B, P, D)
